# baseline (speedup 1.0000x reference)
import jax
import jax.numpy as jnp
from jax.experimental import pallas as pl
from jax.experimental.pallas import tpu as pltpu

_BLOCK_B = 8
_GRID = 64 // _BLOCK_B


def _noising_kernel(t_ref, betas_bar_ref, x_ref, noise_ref, noise_full_ref,
                    out_ref, noise_out_ref, sems):
    i = pl.program_id(0)
    src = noise_full_ref.at[pl.ds(i * _BLOCK_B, _BLOCK_B)]
    dst = noise_out_ref.at[pl.ds(i * _BLOCK_B, _BLOCK_B)]
    pltpu.make_async_copy(src, dst, sems.at[i]).start()

    t = t_ref[0]
    beta = betas_bar_ref[t, 0]
    sa = jnp.sqrt(1.0 - beta)
    sb = jnp.sqrt(beta)
    out_ref[...] = sa * x_ref[...] + sb * noise_ref[...]

    @pl.when(i == pl.num_programs(0) - 1)
    def _wait_all():
        for j in range(_GRID):
            s = noise_full_ref.at[pl.ds(j * _BLOCK_B, _BLOCK_B)]
            d = noise_out_ref.at[pl.ds(j * _BLOCK_B, _BLOCK_B)]
            pltpu.make_async_copy(s, d, sems.at[j]).wait()


def kernel(x, t, betas_bar, noise):
    t_arr = jnp.asarray(t, dtype=jnp.int32).reshape((1,))
    b, c, h, w = x.shape
    blk = (_BLOCK_B, c, h, w)
    noised, noise_out = pl.pallas_call(
        _noising_kernel,
        grid=(b // _BLOCK_B,),
        in_specs=[
            pl.BlockSpec(memory_space=pltpu.SMEM),
            pl.BlockSpec(memory_space=pltpu.SMEM),
            pl.BlockSpec(blk, lambda i: (i, 0, 0, 0)),
            pl.BlockSpec(blk, lambda i: (i, 0, 0, 0)),
            pl.BlockSpec(memory_space=pltpu.MemorySpace.HBM),
        ],
        out_specs=[
            pl.BlockSpec(blk, lambda i: (i, 0, 0, 0)),
            pl.BlockSpec(memory_space=pltpu.MemorySpace.HBM),
        ],
        out_shape=[
            jax.ShapeDtypeStruct(x.shape, x.dtype),
            jax.ShapeDtypeStruct(x.shape, x.dtype),
        ],
        scratch_shapes=[pltpu.SemaphoreType.DMA((_GRID,))],
    )(t_arr, betas_bar, x, noise, noise)
    return noised, noise_out


# speedup vs baseline: 14.9826x; 14.9826x over previous
import functools

import jax
import jax.numpy as jnp
from jax import lax
from jax.experimental import pallas as pl
from jax.experimental.pallas import tpu as pltpu
from jax.experimental.pallas import tpu_sc as plsc

_BLOCK_B = 8
_ROWS = 43008
_W = 224
_NWORKERS = 32
_ROWS_PER_WORKER = _ROWS // _NWORKERS
_CHUNKS = 8
_CHUNK_ROWS = _ROWS_PER_WORKER // _CHUNKS


def _noising_kernel(t_ref, betas_bar_ref, x_ref, noise_ref, out_ref):
    t = t_ref[0]
    beta = betas_bar_ref[t, 0]
    sa = jnp.sqrt(1.0 - beta)
    sb = jnp.sqrt(beta)
    out_ref[...] = sa * x_ref[...] + sb * noise_ref[...]


def _sc_copy_kernel(src_hbm, dst_hbm, buf0, buf1, in_s0, in_s1, out_s0, out_s1):
    wid = lax.axis_index("s") * 2 + lax.axis_index("c")
    base = wid * _ROWS_PER_WORKER
    bufs = (buf0, buf1)
    in_sems = (in_s0, in_s1)
    out_sems = (out_s0, out_s1)

    def in_copy(k):
        return pltpu.make_async_copy(
            src_hbm.at[pl.ds(base + k * _CHUNK_ROWS, _CHUNK_ROWS)],
            bufs[k % 2], in_sems[k % 2])

    def out_copy(k):
        return pltpu.make_async_copy(
            bufs[k % 2],
            dst_hbm.at[pl.ds(base + k * _CHUNK_ROWS, _CHUNK_ROWS)],
            out_sems[k % 2])

    in_copy(0).start()
    for k in range(_CHUNKS):
        in_copy(k).wait()
        out_copy(k).start()
        if k + 1 < _CHUNKS:
            if k >= 1:
                out_copy(k - 1).wait()
            in_copy(k + 1).start()
    out_copy(_CHUNKS - 2).wait()
    out_copy(_CHUNKS - 1).wait()


def kernel(x, t, betas_bar, noise):
    t_arr = jnp.asarray(t, dtype=jnp.int32).reshape((1,))
    b, c, h, w = x.shape
    blk = (_BLOCK_B, c, h, w)
    noised = pl.pallas_call(
        _noising_kernel,
        grid=(b // _BLOCK_B,),
        in_specs=[
            pl.BlockSpec(memory_space=pltpu.SMEM),
            pl.BlockSpec(memory_space=pltpu.SMEM),
            pl.BlockSpec(blk, lambda i: (i, 0, 0, 0)),
            pl.BlockSpec(blk, lambda i: (i, 0, 0, 0)),
        ],
        out_specs=pl.BlockSpec(blk, lambda i: (i, 0, 0, 0)),
        out_shape=jax.ShapeDtypeStruct(x.shape, x.dtype),
    )(t_arr, betas_bar, x, noise)

    sc_copy = pl.kernel(
        _sc_copy_kernel,
        out_type=jax.ShapeDtypeStruct((_ROWS, _W), noise.dtype),
        mesh=plsc.VectorSubcoreMesh(core_axis_name="c", subcore_axis_name="s"),
        scratch_types=[
            pltpu.VMEM((_CHUNK_ROWS, _W), noise.dtype),
            pltpu.VMEM((_CHUNK_ROWS, _W), noise.dtype),
            pltpu.SemaphoreType.DMA,
            pltpu.SemaphoreType.DMA,
            pltpu.SemaphoreType.DMA,
            pltpu.SemaphoreType.DMA,
        ],
        compiler_params=pltpu.CompilerParams(use_tc_tiling_on_sc=True),
    )
    noise_out = sc_copy(noise.reshape(_ROWS, _W)).reshape(x.shape)
    return noised, noise_out


# speedup vs baseline: 20.2414x; 1.3510x over previous
import jax
import jax.numpy as jnp
from jax import lax
from jax.experimental import pallas as pl
from jax.experimental.pallas import tpu as pltpu

_ROWS = 43008
_W = 224
_NCHUNK = 16
_CR = _ROWS // _NCHUNK
_D = 4


def _ring_kernel(t_ref, betas_bar_ref, x_hbm, n_hbm, y_hbm, ny_hbm,
                 xbuf, nbuf, in_sems, out_sems):
    i = pl.program_id(0)

    def in_cp(k, slot):
        return (pltpu.make_async_copy(x_hbm.at[pl.ds(k * _CR, _CR)],
                                      xbuf.at[slot], in_sems.at[slot]),
                pltpu.make_async_copy(n_hbm.at[pl.ds(k * _CR, _CR)],
                                      nbuf.at[slot], in_sems.at[slot]))

    def out_cp(k, slot):
        return (pltpu.make_async_copy(xbuf.at[slot],
                                      y_hbm.at[pl.ds(k * _CR, _CR)],
                                      out_sems.at[slot]),
                pltpu.make_async_copy(nbuf.at[slot],
                                      ny_hbm.at[pl.ds(k * _CR, _CR)],
                                      out_sems.at[slot]))

    @pl.when(i == 0)
    def _prologue():
        for k in range(_D - 1):
            a, b = in_cp(k, k)
            a.start()
            b.start()

    t = t_ref[0]
    beta = betas_bar_ref[t, 0]
    sa = jnp.sqrt(1.0 - beta)
    sb = jnp.sqrt(beta)

    for slot in range(_D):
        @pl.when(lax.rem(i, _D) == slot)
        def _step(slot=slot):
            a, b = in_cp(i, slot)
            a.wait()
            b.wait()
            xbuf[slot] = sa * xbuf[slot] + sb * nbuf[slot]
            ya, yb = out_cp(i, slot)
            ya.start()
            yb.start()

    j = i + _D - 1

    @pl.when(j < _NCHUNK)
    def _refill():
        @pl.when(i >= 1)
        def _drain_prev():
            for slot in range(_D):
                @pl.when(lax.rem(i - 1, _D) == slot)
                def _w(slot=slot):
                    oa, ob = out_cp(i - 1, slot)
                    oa.wait()
                    ob.wait()

        for slot in range(_D):
            @pl.when(lax.rem(j, _D) == slot)
            def _s(slot=slot):
                a, b = in_cp(j, slot)
                a.start()
                b.start()

    @pl.when(i == _NCHUNK - 1)
    def _epilogue():
        for k in range(_NCHUNK - _D, _NCHUNK):
            oa, ob = out_cp(k, k % _D)
            oa.wait()
            ob.wait()


def kernel(x, t, betas_bar, noise):
    t_arr = jnp.asarray(t, dtype=jnp.int32).reshape((1,))
    x2 = x.reshape(_ROWS, _W)
    n2 = noise.reshape(_ROWS, _W)
    noised, noise_out = pl.pallas_call(
        _ring_kernel,
        grid=(_NCHUNK,),
        in_specs=[
            pl.BlockSpec(memory_space=pltpu.SMEM),
            pl.BlockSpec(memory_space=pltpu.SMEM),
            pl.BlockSpec(memory_space=pltpu.MemorySpace.HBM),
            pl.BlockSpec(memory_space=pltpu.MemorySpace.HBM),
        ],
        out_specs=[
            pl.BlockSpec(memory_space=pltpu.MemorySpace.HBM),
            pl.BlockSpec(memory_space=pltpu.MemorySpace.HBM),
        ],
        out_shape=[
            jax.ShapeDtypeStruct((_ROWS, _W), x.dtype),
            jax.ShapeDtypeStruct((_ROWS, _W), x.dtype),
        ],
        scratch_shapes=[
            pltpu.VMEM((_D, _CR, _W), x.dtype),
            pltpu.VMEM((_D, _CR, _W), x.dtype),
            pltpu.SemaphoreType.DMA((_D,)),
            pltpu.SemaphoreType.DMA((_D,)),
        ],
    )(t_arr, betas_bar, x2, n2)
    return noised.reshape(x.shape), noise_out.reshape(x.shape)
